# baseline (speedup 1.0000x reference)
import functools

import jax
import jax.numpy as jnp
from jax import lax
from jax.experimental import pallas as pl
from jax.experimental.pallas import tpu as pltpu
from jax.experimental.pallas import tpu_sc as plsc

N = 10000
D = 128
NC = 2
NS = 16
NW = NC * NS
K = 160

NP = 10240
ROWS_PER_TILE = NP // NS


def _edge_pass_body(nl, h_hbm, src_hbm, dst_hbm, w_hbm, out_hbm,
                    acc, rowsA, rowsB, sidxA, sidxB, didxA, didxB,
                    wA, wB, gsemA, gsemB, isemA, isemB):
    c = lax.axis_index("c")
    s = lax.axis_index("s")
    tb = (c * NS + s) * (nl * K)

    zv = jnp.zeros((16,), jnp.float32)

    def zfill(j, _):
        for g in range(8):
            rowsA[j, pl.ds(g * 16, 16)] = zv
        return 0
    lax.fori_loop(0, K, zfill, 0)

    r0 = s * ROWS_PER_TILE

    def zcopy(i, _):
        pltpu.sync_copy(rowsA, acc.at[pl.ds(r0 + i * K, K)])
        return 0
    lax.fori_loop(0, ROWS_PER_TILE // K, zcopy, 0)

    plsc.subcore_barrier()

    def scale_scatter(rows, wbuf, didx):
        def scale(jb, _):
            wv = wbuf[pl.ds(jb * 16, 16)]
            for l in range(16):
                wj = wv[l]
                j = jb * 16 + l
                for g in range(8):
                    rows[j, pl.ds(g * 16, 16)] = rows[j, pl.ds(g * 16, 16)] * wj
            return 0
        lax.fori_loop(0, K // 16, scale, 0)
        pltpu.sync_copy(rows, acc.at[didx], add=True)

    def issue_idx(ci, sidx, didx, wbuf, isem):
        b = tb + ci * K
        pltpu.async_copy(src_hbm.at[pl.ds(b, K)], sidx, isem)
        pltpu.async_copy(dst_hbm.at[pl.ds(b, K)], didx, isem)
        pltpu.async_copy(w_hbm.at[pl.ds(b, K)], wbuf, isem)

    def wait_idx(ci, sidx, didx, wbuf, isem):
        b = tb + ci * K
        pltpu.make_async_copy(src_hbm.at[pl.ds(b, K)], sidx, isem).wait()
        pltpu.make_async_copy(dst_hbm.at[pl.ds(b, K)], didx, isem).wait()
        pltpu.make_async_copy(w_hbm.at[pl.ds(b, K)], wbuf, isem).wait()

    issue_idx(0, sidxA, didxA, wA, isemA)
    wait_idx(0, sidxA, didxA, wA, isemA)
    pltpu.async_copy(h_hbm.at[sidxA], rowsA, gsemA)
    issue_idx(1, sidxB, didxB, wB, isemB)

    def piped(i, _):
        ca = 2 * i
        pltpu.make_async_copy(h_hbm.at[sidxA], rowsA, gsemA).wait()
        wait_idx(ca + 1, sidxB, didxB, wB, isemB)
        pltpu.async_copy(h_hbm.at[sidxB], rowsB, gsemB)
        scale_scatter(rowsA, wA, didxA)

        @pl.when(ca + 2 < nl)
        def _():
            issue_idx(ca + 2, sidxA, didxA, wA, isemA)
        pltpu.make_async_copy(h_hbm.at[sidxB], rowsB, gsemB).wait()

        @pl.when(ca + 2 < nl)
        def _():
            wait_idx(ca + 2, sidxA, didxA, wA, isemA)
            pltpu.async_copy(h_hbm.at[sidxA], rowsA, gsemA)
        scale_scatter(rowsB, wB, didxB)

        @pl.when(ca + 3 < nl)
        def _():
            issue_idx(ca + 3, sidxB, didxB, wB, isemB)
        return 0
    lax.fori_loop(0, nl // 2, piped, 0)

    plsc.subcore_barrier()

    def wrow(i, _):
        sl = pl.ds(r0 + i * K, K)
        pltpu.sync_copy(acc.at[sl], out_hbm.at[c].at[sl])
        return 0
    lax.fori_loop(0, ROWS_PER_TILE // K, wrow, 0)


@functools.partial(jax.jit, static_argnames=("nl",))
def _edge_pass(h, src, dst, w, nl):
    mesh = plsc.VectorSubcoreMesh(core_axis_name="c", subcore_axis_name="s")
    body = functools.partial(_edge_pass_body, nl)
    return pl.kernel(
        body,
        out_type=jax.ShapeDtypeStruct((NC, NP, D), jnp.float32),
        mesh=mesh,
        scratch_types=[
            pltpu.VMEM_SHARED((NP, D), jnp.float32),
            pltpu.VMEM((K, D), jnp.float32),
            pltpu.VMEM((K, D), jnp.float32),
            pltpu.VMEM((K,), jnp.int32),
            pltpu.VMEM((K,), jnp.int32),
            pltpu.VMEM((K,), jnp.int32),
            pltpu.VMEM((K,), jnp.int32),
            pltpu.VMEM((K,), jnp.float32),
            pltpu.VMEM((K,), jnp.float32),
            pltpu.SemaphoreType.DMA,
            pltpu.SemaphoreType.DMA,
            pltpu.SemaphoreType.DMA,
            pltpu.SemaphoreType.DMA,
        ],
    )(h, src, dst, w)


KD = 1280
KP = 640
ALN3 = NS * 2 * KD
DSTK = 3 * NP
DSL = DSTK // NS


def _rsqrt16(x):
    y = 0.5 * (x + 1.0)
    for _ in range(13):
        y = 0.5 * (y + x / y)
    return 1.0 / y


def _norm_pass_body(nchunks, rid_hbm, cid_hbm, ew_hbm, norm_hbm, dinv_hbm,
                    deg, dsl, cdA, cdB, wdA, wdB, rA, rB, cA, cB,
                    w3A, w3B, aA, aB, bA, bB, oA, oB,
                    dsemA, dsemB, isemA, isemB, gsemA, gsemB):
    c = lax.axis_index("c")
    s = lax.axis_index("s")

    zv = jnp.zeros((16,), jnp.float32)

    def zfill(j, _):
        dsl[pl.ds(j * 16, 16)] = zv
        return 0
    lax.fori_loop(0, DSL // 16, zfill, 0)
    pltpu.sync_copy(dsl, deg.at[pl.ds(s * DSL, DSL)])
    plsc.subcore_barrier()

    nd = nchunks * 2
    tb1 = s * (nd * KD)

    def d_issue(ci, cb, wb, sem):
        b = tb1 + ci * KD
        pltpu.async_copy(cid_hbm.at[pl.ds(b, KD)], cb, sem)
        pltpu.async_copy(ew_hbm.at[pl.ds(b, KD)], wb, sem)

    def d_wait(ci, cb, wb, sem):
        b = tb1 + ci * KD
        pltpu.make_async_copy(cid_hbm.at[pl.ds(b, KD)], cb, sem).wait()
        pltpu.make_async_copy(ew_hbm.at[pl.ds(b, KD)], wb, sem).wait()

    d_issue(0, cdA, wdA, dsemA)
    d_issue(1, cdB, wdB, dsemB)

    def dloop(i, _):
        ca = 2 * i
        d_wait(ca, cdA, wdA, dsemA)
        pltpu.sync_copy(wdA, deg.at[cdA], add=True)

        @pl.when(ca + 2 < nd)
        def _():
            d_issue(ca + 2, cdA, wdA, dsemA)
        d_wait(ca + 1, cdB, wdB, dsemB)
        pltpu.sync_copy(wdB, deg.at[cdB], add=True)

        @pl.when(ca + 3 < nd)
        def _():
            d_issue(ca + 3, cdB, wdB, dsemB)
        return 0
    lax.fori_loop(0, nd // 2, dloop, 0)
    plsc.subcore_barrier()

    pltpu.sync_copy(deg.at[pl.ds(s * DSL, DSL)], dsl)

    def dinvloop(j, _):
        dv = dsl[pl.ds(j * 16, 16)]
        dsl[pl.ds(j * 16, 16)] = _rsqrt16(dv + 1.0)
        return 0
    lax.fori_loop(0, DSL // 16, dinvloop, 0)
    pltpu.sync_copy(dsl, dinv_hbm.at[pl.ds(s * DSL, DSL)])
    plsc.subcore_barrier()

    n3 = nchunks * 2
    tb3 = (c * NS + s) * (n3 * KP)

    def p_issue_idx(ci, rb, cb, wb, sem):
        b = tb3 + ci * KP
        pltpu.async_copy(rid_hbm.at[pl.ds(b, KP)], rb, sem)
        pltpu.async_copy(cid_hbm.at[pl.ds(b, KP)], cb, sem)
        pltpu.async_copy(ew_hbm.at[pl.ds(b, KP)], wb, sem)

    def p_wait_idx(ci, rb, cb, wb, sem):
        b = tb3 + ci * KP
        pltpu.make_async_copy(rid_hbm.at[pl.ds(b, KP)], rb, sem).wait()
        pltpu.make_async_copy(cid_hbm.at[pl.ds(b, KP)], cb, sem).wait()
        pltpu.make_async_copy(ew_hbm.at[pl.ds(b, KP)], wb, sem).wait()

    def p_issue_g(rb, cb, ab, bb, sem):
        pltpu.async_copy(dinv_hbm.at[rb], ab, sem)
        pltpu.async_copy(dinv_hbm.at[cb], bb, sem)

    def p_wait_g(rb, cb, ab, bb, sem):
        pltpu.make_async_copy(dinv_hbm.at[rb], ab, sem).wait()
        pltpu.make_async_copy(dinv_hbm.at[cb], bb, sem).wait()

    def p_vec(ci, wb, ab, bb, ob):
        def nvec(j, _):
            sl = pl.ds(j * 16, 16)
            ob[sl] = ab[sl] * wb[sl] * bb[sl]
            return 0
        lax.fori_loop(0, KP // 16, nvec, 0)
        pltpu.sync_copy(ob, norm_hbm.at[pl.ds(tb3 + ci * KP, KP)])

    p_issue_idx(0, rA, cA, w3A, isemA)
    p_wait_idx(0, rA, cA, w3A, isemA)
    p_issue_g(rA, cA, aA, bA, gsemA)
    p_issue_idx(1, rB, cB, w3B, isemB)

    def nloop(i, _):
        ca = 2 * i
        p_wait_g(rA, cA, aA, bA, gsemA)
        p_wait_idx(ca + 1, rB, cB, w3B, isemB)
        p_issue_g(rB, cB, aB, bB, gsemB)
        p_vec(ca, w3A, aA, bA, oA)

        @pl.when(ca + 2 < n3)
        def _():
            p_issue_idx(ca + 2, rA, cA, w3A, isemA)
        p_wait_g(rB, cB, aB, bB, gsemB)

        @pl.when(ca + 2 < n3)
        def _():
            p_wait_idx(ca + 2, rA, cA, w3A, isemA)
            p_issue_g(rA, cA, aA, bA, gsemA)
        p_vec(ca + 1, w3B, aB, bB, oB)

        @pl.when(ca + 3 < n3)
        def _():
            p_issue_idx(ca + 3, rB, cB, w3B, isemB)
        return 0
    lax.fori_loop(0, n3 // 2, nloop, 0)


@functools.partial(jax.jit, static_argnames=("nchunks",))
def _norm_pass(rid, cid, ew, nchunks):
    mesh = plsc.VectorSubcoreMesh(core_axis_name="c", subcore_axis_name="s")
    body = functools.partial(_norm_pass_body, nchunks)
    return pl.kernel(
        body,
        out_type=[jax.ShapeDtypeStruct(rid.shape, jnp.float32),
                  jax.ShapeDtypeStruct((DSTK,), jnp.float32)],
        mesh=mesh,
        scratch_types=[
            pltpu.VMEM_SHARED((DSTK,), jnp.float32),
            pltpu.VMEM((DSL,), jnp.float32),
            pltpu.VMEM((KD,), jnp.int32),
            pltpu.VMEM((KD,), jnp.int32),
            pltpu.VMEM((KD,), jnp.float32),
            pltpu.VMEM((KD,), jnp.float32),
            pltpu.VMEM((KP,), jnp.int32),
            pltpu.VMEM((KP,), jnp.int32),
            pltpu.VMEM((KP,), jnp.int32),
            pltpu.VMEM((KP,), jnp.int32),
            pltpu.VMEM((KP,), jnp.float32),
            pltpu.VMEM((KP,), jnp.float32),
            pltpu.VMEM((KP,), jnp.float32),
            pltpu.VMEM((KP,), jnp.float32),
            pltpu.VMEM((KP,), jnp.float32),
            pltpu.VMEM((KP,), jnp.float32),
            pltpu.VMEM((KP,), jnp.float32),
            pltpu.VMEM((KP,), jnp.float32),
            pltpu.SemaphoreType.DMA,
            pltpu.SemaphoreType.DMA,
            pltpu.SemaphoreType.DMA,
            pltpu.SemaphoreType.DMA,
            pltpu.SemaphoreType.DMA,
            pltpu.SemaphoreType.DMA,
        ],
    )(rid, cid, ew)


def _seg(h, src, dst, w):
    et = src.shape[0]
    align = 2 * NW * K
    etp = -(-et // align) * align
    pad = etp - et
    if pad:
        sp = (jnp.arange(pad, dtype=src.dtype) % N)
        src = jnp.concatenate([src, sp])
        dst = jnp.concatenate([dst, sp])
        w = jnp.concatenate([w, jnp.zeros((pad,), w.dtype)])
    p = _edge_pass(h, src, dst, w, etp // (NW * K))
    return p[0, :N] + p[1, :N]


def kernel(x, edge_index, edge_in, in_w, edge_out, out_w, edge_index2,
           edge_weight, edge_weight2, lin1_w, lin2_w, ib1_ln_w, ib1_ln_b,
           ib1_c1_w, ib1_c1_b, ib1_c2_w, ib1_c2_b, ib2_ln_w, ib2_ln_b,
           ib2_c1_w, ib2_c1_b, ib2_c2_w, ib2_c2_b, conv1_w, conv1_b):
    r1, c1 = edge_index[0], edge_index[1]
    r2, c2 = edge_in[0], edge_in[1]
    r3, c3 = edge_out[0], edge_out[1]
    e3 = 3 * r1.shape[0]
    nchunks = -(-e3 // ALN3)
    pad3 = nchunks * ALN3 - e3
    zi = jnp.arange(pad3, dtype=r1.dtype) % N
    zf = jnp.zeros((pad3,), jnp.float32)
    ones = jnp.ones(r1.shape, jnp.float32)
    rid = jnp.concatenate([r1, r2 + NP, r3 + 2 * NP, zi])
    cid = jnp.concatenate([c1, c2 + NP, c3 + 2 * NP, zi])
    ew_all = jnp.concatenate([ones, in_w, out_w, zf])
    norm_all, dinv_s = _norm_pass(rid, cid, ew_all, nchunks)
    d1 = dinv_s[:N]
    d2 = dinv_s[NP:NP + N]
    d3 = dinv_s[2 * NP:2 * NP + N]
    dsum = (d1 * d1 + d2 * d2 + d3 * d3)[:, None]
    rows_all = jnp.concatenate([r1, r2, r3, zi])
    cols_all = jnp.concatenate([c1, c2, c3, zi])

    src, dst = edge_index[0], edge_index[1]
    src2, dst2 = edge_index2[0], edge_index2[1]
    src12 = jnp.concatenate([src, src2 + N])
    dst12 = jnp.concatenate([dst, dst2])
    w12 = jnp.concatenate([edge_weight, edge_weight2])

    symx = x @ lin1_w
    symx = _seg(symx, rows_all, cols_all, norm_all) + dsum * symx
    x0 = x @ ib1_ln_w + ib1_ln_b
    x12 = (_seg(x @ ib1_c1_w, src, dst, edge_weight) +
           _seg(x @ ib1_c2_w, src2, dst2, edge_weight2))
    x12 = x12 + ib1_c1_b + ib1_c2_b
    h = jnp.concatenate([x0 + x12, symx], axis=-1)
    h = jax.nn.relu(h @ conv1_w.T + conv1_b)
    x0 = h @ ib2_ln_w + ib2_ln_b
    x12 = (_seg(h @ ib2_c1_w, src, dst, edge_weight) +
           _seg(h @ ib2_c2_w, src2, dst2, edge_weight2))
    h = jax.nn.relu(x0 + x12 + ib2_c1_b + ib2_c2_b)
    symx = h @ lin2_w
    return _seg(symx, rows_all, cols_all, norm_all) + dsum * symx


# speedup vs baseline: 1.0276x; 1.0276x over previous
import functools

import jax
import jax.numpy as jnp
from jax import lax
from jax.experimental import pallas as pl
from jax.experimental.pallas import tpu as pltpu
from jax.experimental.pallas import tpu_sc as plsc

N = 10000
D = 128
NC = 2
NS = 16
NW = NC * NS
K = 160

NP = 10240
ROWS_PER_TILE = NP // NS


def _edge_pass_body(nl, h_hbm, src_hbm, dst_hbm, w_hbm, out_hbm,
                    acc, rowsA, rowsB, sidxA, sidxB, didxA, didxB,
                    wA, wB, gsemA, gsemB, isemA, isemB):
    c = lax.axis_index("c")
    s = lax.axis_index("s")
    tb = (c * NS + s) * (nl * K)

    zv = jnp.zeros((16,), jnp.float32)

    def zfill(j, _):
        for g in range(8):
            rowsA[j, pl.ds(g * 16, 16)] = zv
        return 0
    lax.fori_loop(0, K, zfill, 0)

    r0 = s * ROWS_PER_TILE

    def zcopy(i, _):
        pltpu.sync_copy(rowsA, acc.at[pl.ds(r0 + i * K, K)])
        return 0
    lax.fori_loop(0, ROWS_PER_TILE // K, zcopy, 0)

    plsc.subcore_barrier()

    def scale_scatter(rows, wbuf, didx):
        def scale(jb, _):
            wv = wbuf[pl.ds(jb * 16, 16)]
            for l in range(16):
                wj = wv[l]
                j = jb * 16 + l
                for g in range(8):
                    rows[j, pl.ds(g * 16, 16)] = rows[j, pl.ds(g * 16, 16)] * wj
            return 0
        lax.fori_loop(0, K // 16, scale, 0)
        pltpu.sync_copy(rows, acc.at[didx], add=True)

    def issue_idx(ci, sidx, didx, wbuf, isem):
        b = tb + ci * K
        pltpu.async_copy(src_hbm.at[pl.ds(b, K)], sidx, isem)
        pltpu.async_copy(dst_hbm.at[pl.ds(b, K)], didx, isem)
        pltpu.async_copy(w_hbm.at[pl.ds(b, K)], wbuf, isem)

    def wait_idx(ci, sidx, didx, wbuf, isem):
        b = tb + ci * K
        pltpu.make_async_copy(src_hbm.at[pl.ds(b, K)], sidx, isem).wait()
        pltpu.make_async_copy(dst_hbm.at[pl.ds(b, K)], didx, isem).wait()
        pltpu.make_async_copy(w_hbm.at[pl.ds(b, K)], wbuf, isem).wait()

    issue_idx(0, sidxA, didxA, wA, isemA)
    wait_idx(0, sidxA, didxA, wA, isemA)
    pltpu.async_copy(h_hbm.at[sidxA], rowsA, gsemA)
    issue_idx(1, sidxB, didxB, wB, isemB)

    def piped(i, _):
        ca = 2 * i
        pltpu.make_async_copy(h_hbm.at[sidxA], rowsA, gsemA).wait()
        wait_idx(ca + 1, sidxB, didxB, wB, isemB)
        pltpu.async_copy(h_hbm.at[sidxB], rowsB, gsemB)
        scale_scatter(rowsA, wA, didxA)

        @pl.when(ca + 2 < nl)
        def _():
            issue_idx(ca + 2, sidxA, didxA, wA, isemA)
        pltpu.make_async_copy(h_hbm.at[sidxB], rowsB, gsemB).wait()

        @pl.when(ca + 2 < nl)
        def _():
            wait_idx(ca + 2, sidxA, didxA, wA, isemA)
            pltpu.async_copy(h_hbm.at[sidxA], rowsA, gsemA)
        scale_scatter(rowsB, wB, didxB)

        @pl.when(ca + 3 < nl)
        def _():
            issue_idx(ca + 3, sidxB, didxB, wB, isemB)
        return 0
    lax.fori_loop(0, nl // 2, piped, 0)

    plsc.subcore_barrier()

    def wrow(i, _):
        sl = pl.ds(r0 + i * K, K)
        pltpu.sync_copy(acc.at[sl], out_hbm.at[c].at[sl])
        return 0
    lax.fori_loop(0, ROWS_PER_TILE // K, wrow, 0)


@functools.partial(jax.jit, static_argnames=("nl",))
def _edge_pass(h, src, dst, w, nl):
    mesh = plsc.VectorSubcoreMesh(core_axis_name="c", subcore_axis_name="s")
    body = functools.partial(_edge_pass_body, nl)
    return pl.kernel(
        body,
        out_type=jax.ShapeDtypeStruct((NC, NP, D), jnp.float32),
        mesh=mesh,
        scratch_types=[
            pltpu.VMEM_SHARED((NP, D), jnp.float32),
            pltpu.VMEM((K, D), jnp.float32),
            pltpu.VMEM((K, D), jnp.float32),
            pltpu.VMEM((K,), jnp.int32),
            pltpu.VMEM((K,), jnp.int32),
            pltpu.VMEM((K,), jnp.int32),
            pltpu.VMEM((K,), jnp.int32),
            pltpu.VMEM((K,), jnp.float32),
            pltpu.VMEM((K,), jnp.float32),
            pltpu.SemaphoreType.DMA,
            pltpu.SemaphoreType.DMA,
            pltpu.SemaphoreType.DMA,
            pltpu.SemaphoreType.DMA,
        ],
    )(h, src, dst, w)


KD = 1280
KP = 640
ALN3 = NS * 2 * KD
DSTK = 3 * NP
DSL = DSTK // NS


def _rsqrt16(x):
    y = 0.5 * (x + 1.0)
    for _ in range(13):
        y = 0.5 * (y + x / y)
    return 1.0 / y


def _norm_pass_body(nchunks, rid_hbm, cid_hbm, ew_hbm, norm_hbm, dinv_hbm,
                    deg, dsl, cdA, cdB, wdA, wdB, rA, rB, cA, cB,
                    w3A, w3B, aA, aB, bA, bB, oA, oB,
                    dsemA, dsemB, isemA, isemB, gsemA, gsemB):
    c = lax.axis_index("c")
    s = lax.axis_index("s")

    zv = jnp.zeros((16,), jnp.float32)

    def zfill(j, _):
        dsl[pl.ds(j * 16, 16)] = zv
        return 0
    lax.fori_loop(0, DSL // 16, zfill, 0)
    pltpu.sync_copy(dsl, deg.at[pl.ds(s * DSL, DSL)])
    plsc.subcore_barrier()

    nd = nchunks * 2
    tb1 = s * (nd * KD)

    def d_issue(ci, cb, wb, sem):
        b = tb1 + ci * KD
        pltpu.async_copy(cid_hbm.at[pl.ds(b, KD)], cb, sem)
        pltpu.async_copy(ew_hbm.at[pl.ds(b, KD)], wb, sem)

    def d_wait(ci, cb, wb, sem):
        b = tb1 + ci * KD
        pltpu.make_async_copy(cid_hbm.at[pl.ds(b, KD)], cb, sem).wait()
        pltpu.make_async_copy(ew_hbm.at[pl.ds(b, KD)], wb, sem).wait()

    d_issue(0, cdA, wdA, dsemA)
    d_issue(1, cdB, wdB, dsemB)

    def dloop(i, _):
        ca = 2 * i
        d_wait(ca, cdA, wdA, dsemA)
        pltpu.sync_copy(wdA, deg.at[cdA], add=True)

        @pl.when(ca + 2 < nd)
        def _():
            d_issue(ca + 2, cdA, wdA, dsemA)
        d_wait(ca + 1, cdB, wdB, dsemB)
        pltpu.sync_copy(wdB, deg.at[cdB], add=True)

        @pl.when(ca + 3 < nd)
        def _():
            d_issue(ca + 3, cdB, wdB, dsemB)
        return 0
    lax.fori_loop(0, nd // 2, dloop, 0)
    plsc.subcore_barrier()

    pltpu.sync_copy(deg.at[pl.ds(s * DSL, DSL)], dsl)

    def dinvloop(j, _):
        dv = dsl[pl.ds(j * 16, 16)]
        dsl[pl.ds(j * 16, 16)] = _rsqrt16(dv + 1.0)
        return 0
    lax.fori_loop(0, DSL // 16, dinvloop, 0)
    pltpu.sync_copy(dsl, dinv_hbm.at[pl.ds(s * DSL, DSL)])
    plsc.subcore_barrier()

    n3 = nchunks * 2
    tb3 = (c * NS + s) * (n3 * KP)

    def p_issue_idx(ci, rb, cb, wb, sem):
        b = tb3 + ci * KP
        pltpu.async_copy(rid_hbm.at[pl.ds(b, KP)], rb, sem)
        pltpu.async_copy(cid_hbm.at[pl.ds(b, KP)], cb, sem)
        pltpu.async_copy(ew_hbm.at[pl.ds(b, KP)], wb, sem)

    def p_wait_idx(ci, rb, cb, wb, sem):
        b = tb3 + ci * KP
        pltpu.make_async_copy(rid_hbm.at[pl.ds(b, KP)], rb, sem).wait()
        pltpu.make_async_copy(cid_hbm.at[pl.ds(b, KP)], cb, sem).wait()
        pltpu.make_async_copy(ew_hbm.at[pl.ds(b, KP)], wb, sem).wait()

    def p_issue_g(rb, cb, ab, bb, sem):
        pltpu.async_copy(dinv_hbm.at[rb], ab, sem)
        pltpu.async_copy(dinv_hbm.at[cb], bb, sem)

    def p_wait_g(rb, cb, ab, bb, sem):
        pltpu.make_async_copy(dinv_hbm.at[rb], ab, sem).wait()
        pltpu.make_async_copy(dinv_hbm.at[cb], bb, sem).wait()

    def p_vec(ci, wb, ab, bb, ob):
        def nvec(j, _):
            sl = pl.ds(j * 16, 16)
            ob[sl] = ab[sl] * wb[sl] * bb[sl]
            return 0
        lax.fori_loop(0, KP // 16, nvec, 0)
        pltpu.sync_copy(ob, norm_hbm.at[pl.ds(tb3 + ci * KP, KP)])

    p_issue_idx(0, rA, cA, w3A, isemA)
    p_wait_idx(0, rA, cA, w3A, isemA)
    p_issue_g(rA, cA, aA, bA, gsemA)
    p_issue_idx(1, rB, cB, w3B, isemB)

    def nloop(i, _):
        ca = 2 * i
        p_wait_g(rA, cA, aA, bA, gsemA)
        p_wait_idx(ca + 1, rB, cB, w3B, isemB)
        p_issue_g(rB, cB, aB, bB, gsemB)
        p_vec(ca, w3A, aA, bA, oA)

        @pl.when(ca + 2 < n3)
        def _():
            p_issue_idx(ca + 2, rA, cA, w3A, isemA)
        p_wait_g(rB, cB, aB, bB, gsemB)

        @pl.when(ca + 2 < n3)
        def _():
            p_wait_idx(ca + 2, rA, cA, w3A, isemA)
            p_issue_g(rA, cA, aA, bA, gsemA)
        p_vec(ca + 1, w3B, aB, bB, oB)

        @pl.when(ca + 3 < n3)
        def _():
            p_issue_idx(ca + 3, rB, cB, w3B, isemB)
        return 0
    lax.fori_loop(0, n3 // 2, nloop, 0)


@functools.partial(jax.jit, static_argnames=("nchunks",))
def _norm_pass(rid, cid, ew, nchunks):
    mesh = plsc.VectorSubcoreMesh(core_axis_name="c", subcore_axis_name="s")
    body = functools.partial(_norm_pass_body, nchunks)
    return pl.kernel(
        body,
        out_type=[jax.ShapeDtypeStruct(rid.shape, jnp.float32),
                  jax.ShapeDtypeStruct((DSTK,), jnp.float32)],
        mesh=mesh,
        scratch_types=[
            pltpu.VMEM_SHARED((DSTK,), jnp.float32),
            pltpu.VMEM((DSL,), jnp.float32),
            pltpu.VMEM((KD,), jnp.int32),
            pltpu.VMEM((KD,), jnp.int32),
            pltpu.VMEM((KD,), jnp.float32),
            pltpu.VMEM((KD,), jnp.float32),
            pltpu.VMEM((KP,), jnp.int32),
            pltpu.VMEM((KP,), jnp.int32),
            pltpu.VMEM((KP,), jnp.int32),
            pltpu.VMEM((KP,), jnp.int32),
            pltpu.VMEM((KP,), jnp.float32),
            pltpu.VMEM((KP,), jnp.float32),
            pltpu.VMEM((KP,), jnp.float32),
            pltpu.VMEM((KP,), jnp.float32),
            pltpu.VMEM((KP,), jnp.float32),
            pltpu.VMEM((KP,), jnp.float32),
            pltpu.VMEM((KP,), jnp.float32),
            pltpu.VMEM((KP,), jnp.float32),
            pltpu.SemaphoreType.DMA,
            pltpu.SemaphoreType.DMA,
            pltpu.SemaphoreType.DMA,
            pltpu.SemaphoreType.DMA,
            pltpu.SemaphoreType.DMA,
            pltpu.SemaphoreType.DMA,
        ],
    )(rid, cid, ew)


def _seg(h, src, dst, w):
    et = src.shape[0]
    align = 2 * NW * K
    etp = -(-et // align) * align
    pad = etp - et
    if pad:
        sp = (jnp.arange(pad, dtype=src.dtype) % N)
        src = jnp.concatenate([src, sp])
        dst = jnp.concatenate([dst, sp])
        w = jnp.concatenate([w, jnp.zeros((pad,), w.dtype)])
    p = _edge_pass(h, src, dst, w, etp // (NW * K))
    return p[0, :N] + p[1, :N]


def kernel(x, edge_index, edge_in, in_w, edge_out, out_w, edge_index2,
           edge_weight, edge_weight2, lin1_w, lin2_w, ib1_ln_w, ib1_ln_b,
           ib1_c1_w, ib1_c1_b, ib1_c2_w, ib1_c2_b, ib2_ln_w, ib2_ln_b,
           ib2_c1_w, ib2_c1_b, ib2_c2_w, ib2_c2_b, conv1_w, conv1_b):
    r1, c1 = edge_index[0], edge_index[1]
    r2, c2 = edge_in[0], edge_in[1]
    r3, c3 = edge_out[0], edge_out[1]
    e3 = 3 * r1.shape[0]
    nchunks = -(-e3 // ALN3)
    pad3 = nchunks * ALN3 - e3
    zi = jnp.arange(pad3, dtype=r1.dtype) % N
    zf = jnp.zeros((pad3,), jnp.float32)
    ones = jnp.ones(r1.shape, jnp.float32)
    rid = jnp.concatenate([r1, r2 + NP, r3 + 2 * NP, zi])
    cid = jnp.concatenate([c1, c2 + NP, c3 + 2 * NP, zi])
    ew_all = jnp.concatenate([ones, in_w, out_w, zf])
    norm_all, dinv_s = _norm_pass(rid, cid, ew_all, nchunks)
    d1 = dinv_s[:N]
    d2 = dinv_s[NP:NP + N]
    d3 = dinv_s[2 * NP:2 * NP + N]
    dsum = (d1 * d1 + d2 * d2 + d3 * d3)[:, None]
    rows_all = jnp.concatenate([r1, r2, r3, zi])
    cols_all = jnp.concatenate([c1, c2, c3, zi])

    src, dst = edge_index[0], edge_index[1]
    src2, dst2 = edge_index2[0], edge_index2[1]
    src12 = jnp.concatenate([src, src2 + N])
    dst12 = jnp.concatenate([dst, dst2])
    w12 = jnp.concatenate([edge_weight, edge_weight2])

    symx = x @ lin1_w
    symx = _seg(symx, rows_all, cols_all, norm_all) + dsum * symx
    x0 = x @ ib1_ln_w + ib1_ln_b
    x12 = _seg(jnp.concatenate([x @ ib1_c1_w, x @ ib1_c2_w]), src12, dst12, w12)
    x12 = x12 + ib1_c1_b + ib1_c2_b
    h = jnp.concatenate([x0 + x12, symx], axis=-1)
    h = jax.nn.relu(h @ conv1_w.T + conv1_b)
    x0 = h @ ib2_ln_w + ib2_ln_b
    x12 = _seg(jnp.concatenate([h @ ib2_c1_w, h @ ib2_c2_w]), src12, dst12, w12)
    h = jax.nn.relu(x0 + x12 + ib2_c1_b + ib2_c2_b)
    symx = h @ lin2_w
    return _seg(symx, rows_all, cols_all, norm_all) + dsum * symx
